# baseline (speedup 1.0000x reference)
import functools

import jax
import jax.numpy as jnp
from jax import lax
from jax.experimental import pallas as pl
from jax.experimental.pallas import tpu as pltpu
from jax.experimental.pallas import tpu_sc as plsc

D_MODEL = 64
NC = 2
NS = 16
NW = NC * NS
CHUNK = 1024


def _sc_gather(n_flat: int):
    b_per_w = n_flat // NW
    n_chunks = b_per_w // CHUNK
    mesh = plsc.VectorSubcoreMesh(core_axis_name="c", subcore_axis_name="s")

    @functools.partial(
        pl.kernel,
        out_type=jax.ShapeDtypeStruct((n_flat, D_MODEL), jnp.float32),
        mesh=mesh,
        scratch_types=[
            pltpu.VMEM((CHUNK,), jnp.int32),
            pltpu.VMEM((CHUNK, D_MODEL), jnp.float32),
            pltpu.SemaphoreType.DMA,
        ],
        compiler_params=pltpu.CompilerParams(use_tc_tiling_on_sc=False),
    )
    def body(table_hbm, idx_hbm, out_hbm, idx_v, rows_v, sem):
        wid = lax.axis_index("s") * NC + lax.axis_index("c")
        base = wid * b_per_w

        def step(c, _):
            off = pl.multiple_of(base + c * CHUNK, CHUNK)
            pltpu.sync_copy(idx_hbm.at[pl.ds(off, CHUNK)], idx_v)
            pltpu.async_copy(table_hbm.at[idx_v], rows_v, sem).wait()
            pltpu.sync_copy(rows_v, out_hbm.at[pl.ds(off, CHUNK)])
            return ()

        lax.fori_loop(0, n_chunks, step, ())

    return body


def kernel(x, table):
    b, l = x.shape
    flat = x.reshape(-1).astype(jnp.int32)
    out = _sc_gather(b * l)(table, flat)
    return out.reshape(b, l, D_MODEL)


# speedup vs baseline: 1.0155x; 1.0155x over previous
import functools

import jax
import jax.numpy as jnp
from jax import lax
from jax.experimental import pallas as pl
from jax.experimental.pallas import tpu as pltpu
from jax.experimental.pallas import tpu_sc as plsc

D_MODEL = 64
NC = 2
NS = 16
NW = NC * NS
CHUNK = 400
NB = 4


def _sc_gather(n_flat: int):
    b_per_w = n_flat // NW
    n_chunks = b_per_w // CHUNK
    mesh = plsc.VectorSubcoreMesh(core_axis_name="c", subcore_axis_name="s")

    @functools.partial(
        pl.kernel,
        out_type=jax.ShapeDtypeStruct((n_flat, D_MODEL), jnp.float32),
        mesh=mesh,
        scratch_types=[
            pltpu.VMEM((b_per_w,), jnp.int32),
            [pltpu.VMEM((CHUNK, D_MODEL), jnp.float32) for _ in range(NB)],
            [pltpu.SemaphoreType.DMA for _ in range(NB)],
            [pltpu.SemaphoreType.DMA for _ in range(NB)],
        ],
        compiler_params=pltpu.CompilerParams(use_tc_tiling_on_sc=False),
    )
    def body(table_hbm, idx_hbm, out_hbm, idx_all, rows, sg, so):
        wid = lax.axis_index("s") * NC + lax.axis_index("c")
        base = wid * b_per_w
        pltpu.sync_copy(idx_hbm.at[pl.ds(base, b_per_w)], idx_all)

        def gather(c, b):
            idx_slice = idx_all.at[pl.ds(c * CHUNK, CHUNK)]
            pltpu.async_copy(table_hbm.at[idx_slice], rows[b], sg[b])

        def wait_gather(b):
            idx_slice = idx_all.at[pl.ds(0, CHUNK)]
            pltpu.make_async_copy(table_hbm.at[idx_slice], rows[b], sg[b]).wait()

        def put(c, b):
            off = pl.multiple_of(base + c * CHUNK, 8)
            pltpu.async_copy(rows[b], out_hbm.at[pl.ds(off, CHUNK)], so[b])

        def wait_put(b):
            off = pl.multiple_of(base, 8)
            pltpu.make_async_copy(rows[b], out_hbm.at[pl.ds(off, CHUNK)], so[b]).wait()

        for b in range(NB):
            gather(b, b)

        def grp(g, _):
            c0 = g * NB
            for b in range(NB):
                c = c0 + b
                wait_gather(b)
                put(c, b)
                wait_put(b)
                gather(c + NB, b)
            return ()

        lax.fori_loop(0, (n_chunks - NB) // NB, grp, ())

        for b in range(NB):
            wait_gather(b)
            put(n_chunks - NB + b, b)
        for b in range(NB):
            wait_put(b)

    return body


def kernel(x, table):
    b, l = x.shape
    flat = x.reshape(-1).astype(jnp.int32)
    out = _sc_gather(b * l)(table, flat)
    return out.reshape(b, l, D_MODEL)


# speedup vs baseline: 1.4374x; 1.4155x over previous
import functools

import jax
import jax.numpy as jnp
from jax import lax
from jax.experimental import pallas as pl
from jax.experimental.pallas import tpu as pltpu
from jax.experimental.pallas import tpu_sc as plsc

D_MODEL = 64
NC = 2
NS = 16
NW = NC * NS
CHUNK = 512


def _sc_gather(n_flat: int):
    b_per_w = n_flat // NW
    n_chunks = b_per_w // CHUNK
    mesh = plsc.VectorSubcoreMesh(core_axis_name="c", subcore_axis_name="s")

    @functools.partial(
        pl.kernel,
        out_type=jax.ShapeDtypeStruct((n_flat, D_MODEL), jnp.float32),
        mesh=mesh,
        scratch_types=[
            pltpu.VMEM((CHUNK,), jnp.int32),
            pltpu.VMEM((CHUNK, D_MODEL), jnp.float32),
            pltpu.SemaphoreType.DMA,
        ],
        compiler_params=pltpu.CompilerParams(use_tc_tiling_on_sc=True),
    )
    def body(table_hbm, idx_hbm, out_hbm, idx_v, rows_v, sem):
        wid = lax.axis_index("s") * NC + lax.axis_index("c")
        base = wid * b_per_w

        def step(c, _):
            off = pl.multiple_of(base + c * CHUNK, CHUNK)
            pltpu.sync_copy(idx_hbm.at[pl.ds(off, CHUNK)], idx_v)

            def grp16(g, _):
                vec = idx_v[pl.ds(g * 16, 16)]
                for lane in range(16):
                    i = vec[lane]
                    pltpu.async_copy(
                        table_hbm.at[pl.ds(i, 1), :],
                        rows_v.at[pl.ds(g * 16 + lane, 1), :],
                        sem,
                    )
                return ()

            lax.fori_loop(0, CHUNK // 16, grp16, ())
            pltpu.make_async_copy(
                table_hbm.at[pl.ds(0, CHUNK), :], rows_v, sem
            ).wait()
            pltpu.sync_copy(rows_v, out_hbm.at[pl.ds(off, CHUNK)])
            return ()

        lax.fori_loop(0, n_chunks, step, ())

    return body


def kernel(x, table):
    b, l = x.shape
    flat = x.reshape(-1).astype(jnp.int32)
    out = _sc_gather(b * l)(table, flat)
    return out.reshape(b, l, D_MODEL)


# speedup vs baseline: 1.5114x; 1.0514x over previous
import functools

import jax
import jax.numpy as jnp
from jax import lax
from jax.experimental import pallas as pl
from jax.experimental.pallas import tpu as pltpu
from jax.experimental.pallas import tpu_sc as plsc

D_MODEL = 64
NC = 2
NS = 16
NW = NC * NS
CHUNK = 256
NB = 2


def _sc_gather(n_flat: int):
    b_per_w = n_flat // NW
    n_chunks = b_per_w // CHUNK
    mesh = plsc.VectorSubcoreMesh(core_axis_name="c", subcore_axis_name="s")

    @functools.partial(
        pl.kernel,
        out_type=jax.ShapeDtypeStruct((n_flat, D_MODEL), jnp.float32),
        mesh=mesh,
        scratch_types=[
            pltpu.VMEM((b_per_w,), jnp.int32),
            [pltpu.VMEM((CHUNK, D_MODEL), jnp.float32) for _ in range(NB)],
            [pltpu.SemaphoreType.DMA for _ in range(NB)],
            [pltpu.SemaphoreType.DMA for _ in range(NB)],
        ],
        compiler_params=pltpu.CompilerParams(use_tc_tiling_on_sc=True),
    )
    def body(table_hbm, idx_hbm, out_hbm, idx_all, rows, sg, so):
        wid = lax.axis_index("s") * NC + lax.axis_index("c")
        base = wid * b_per_w
        pltpu.sync_copy(idx_hbm.at[pl.ds(base, b_per_w)], idx_all)

        def gather(c, b):
            def grp16(g, _):
                vec = idx_all[pl.ds(c * CHUNK + g * 16, 16)]
                for lane in range(16):
                    i = vec[lane]
                    pltpu.async_copy(
                        table_hbm.at[pl.ds(i, 1), :],
                        rows[b].at[pl.ds(g * 16 + lane, 1), :],
                        sg[b],
                    )
                return ()

            lax.fori_loop(0, CHUNK // 16, grp16, ())

        def wait_gather(b):
            pltpu.make_async_copy(
                table_hbm.at[pl.ds(0, CHUNK), :], rows[b], sg[b]
            ).wait()

        def put(c, b):
            off = pl.multiple_of(base + c * CHUNK, 8)
            pltpu.async_copy(rows[b], out_hbm.at[pl.ds(off, CHUNK)], so[b])

        def wait_put(b):
            off = pl.multiple_of(base, 8)
            pltpu.make_async_copy(
                rows[b], out_hbm.at[pl.ds(off, CHUNK)], so[b]
            ).wait()

        gather(0, 0)

        def pipe_pair(g, _):
            for k in range(2):
                c = g * 2 + 1 + k
                b = (1 + k) % 2
                gather(c, b)
                wait_gather(1 - b)
                put(c - 1, 1 - b)
                wait_put(1 - b)
            return ()

        lax.fori_loop(0, (n_chunks - 2) // 2, pipe_pair, ())

        c_last = n_chunks - 1
        gather(c_last, c_last % 2)
        wait_gather((c_last - 1) % 2)
        put(c_last - 1, (c_last - 1) % 2)
        wait_put((c_last - 1) % 2)
        wait_gather(c_last % 2)
        put(c_last, c_last % 2)
        wait_put(c_last % 2)

    return body


def kernel(x, table):
    b, l = x.shape
    flat = x.reshape(-1).astype(jnp.int32)
    out = _sc_gather(b * l)(table, flat)
    return out.reshape(b, l, D_MODEL)


# speedup vs baseline: 1.5210x; 1.0063x over previous
import functools

import jax
import jax.numpy as jnp
from jax import lax
from jax.experimental import pallas as pl
from jax.experimental.pallas import tpu as pltpu
from jax.experimental.pallas import tpu_sc as plsc

D_MODEL = 64
NC = 2
NS = 16
NW = NC * NS
CHUNK = 256
NB = 3


def _sc_gather(n_flat: int):
    b_per_w = n_flat // NW
    n_chunks = b_per_w // CHUNK
    mesh = plsc.VectorSubcoreMesh(core_axis_name="c", subcore_axis_name="s")

    @functools.partial(
        pl.kernel,
        out_type=jax.ShapeDtypeStruct((n_flat, D_MODEL), jnp.float32),
        mesh=mesh,
        scratch_types=[
            pltpu.VMEM((b_per_w,), jnp.int32),
            [pltpu.VMEM((CHUNK, D_MODEL), jnp.float32) for _ in range(NB)],
            [pltpu.SemaphoreType.DMA for _ in range(NB)],
            [pltpu.SemaphoreType.DMA for _ in range(NB)],
        ],
        compiler_params=pltpu.CompilerParams(use_tc_tiling_on_sc=True),
    )
    def body(table_hbm, idx_hbm, out_hbm, idx_all, rows, sg, so):
        wid = lax.axis_index("s") * NC + lax.axis_index("c")
        base = wid * b_per_w
        pltpu.sync_copy(idx_hbm.at[pl.ds(base, b_per_w)], idx_all)

        def gather(c, b):
            def grp16(g, _):
                vec = idx_all[pl.ds(c * CHUNK + g * 16, 16)]
                for lane in range(16):
                    i = vec[lane]
                    pltpu.async_copy(
                        table_hbm.at[pl.ds(i, 1), :],
                        rows[b].at[pl.ds(g * 16 + lane, 1), :],
                        sg[b],
                    )
                return ()

            lax.fori_loop(0, CHUNK // 16, grp16, ())

        def wait_gather(b):
            pltpu.make_async_copy(
                table_hbm.at[pl.ds(0, CHUNK), :], rows[b], sg[b]
            ).wait()

        def put(c, b):
            off = pl.multiple_of(base + c * CHUNK, 8)
            pltpu.async_copy(rows[b], out_hbm.at[pl.ds(off, CHUNK)], so[b])

        def wait_put(b):
            off = pl.multiple_of(base, 8)
            pltpu.make_async_copy(
                rows[b], out_hbm.at[pl.ds(off, CHUNK)], so[b]
            ).wait()

        gather(0, 0)
        gather(1, 1)
        wait_gather(0)
        put(0, 0)
        gather(2, 2)
        wait_gather(1)
        put(1, 1)
        wait_put(0)
        gather(3, 0)
        wait_gather(2)
        put(2, 2)

        def pipe3(g, _):
            for k in range(3):
                c = g * 3 + 4 + k
                b = (1 + k) % 3
                wait_put(b)
                gather(c, b)
                wait_gather(k % 3)
                put(c - 1, k % 3)
            return ()

        lax.fori_loop(0, (n_chunks - 4) // 3, pipe3, ())

        c_last = n_chunks - 1
        wait_gather(c_last % 3)
        put(c_last, c_last % 3)
        for b in range(NB):
            wait_put(b)

    return body


def kernel(x, table):
    b, l = x.shape
    flat = x.reshape(-1).astype(jnp.int32)
    out = _sc_gather(b * l)(table, flat)
    return out.reshape(b, l, D_MODEL)
